# baseline (speedup 1.0000x reference)
import functools

import jax
import jax.numpy as jnp
import numpy as np
from jax import lax
from jax.experimental import pallas as pl
from jax.experimental.pallas import tpu as pltpu
from jax.experimental.pallas import tpu_sc as plsc

Hr, Wr = 64, 2048
Hb, Wb = 256, 256
R_MAX = 51.2

NC, NS, L = 2, 16, 16
NW = NC * NS
PLANES = 4 * 64
PPW = PLANES // NW
CHUNKS = Wr // L
TPC = Wr // 128


def _constants():
    col = np.arange(Wr, dtype=np.float64)
    phi = (Wr - 1 - col) / (Wr - 1) * 2.0 * np.pi
    x = R_MAX * np.cos(phi)
    y = R_MAX * np.sin(phi)
    idx_x = x / R_MAX * (Wb / 2 - 0.5) + (Wb / 2 - 0.5)
    idx_y = y / R_MAX * (Hb / 2 - 0.5) + (Hb / 2 - 0.5)
    gx = (idx_x / Wb * 2.0 - 1.0).astype(np.float32)
    gy = (idx_y / Hb * 2.0 - 1.0).astype(np.float32)
    one, half = np.float32(1.0), np.float32(0.5)
    ix = (gx + one) * half * np.float32(Wb - 1)
    iy = (gy + one) * half * np.float32(Hb - 1)
    ix0 = np.floor(ix)
    iy0 = np.floor(iy)
    wx1 = ix - ix0
    wx0 = one - wx1
    wy1 = iy - iy0
    wy0 = one - wy1
    ix0i = ix0.astype(np.int32)
    iy0i = iy0.astype(np.int32)
    idx = np.stack([iy0i, iy0i + 1, ix0i, ix0i + 1])
    w = np.stack([wy0 * wx0, wy0 * wx1, wy1 * wx0, wy1 * wx1])
    return idx.astype(np.int32), w.astype(np.float32)


_IDX, _W = _constants()
_MESH = plsc.VectorSubcoreMesh(core_axis_name="c", subcore_axis_name="s")


@functools.partial(
    pl.kernel,
    out_type=jax.ShapeDtypeStruct((PLANES, Hr // 8, TPC, 8, 128), jnp.float32),
    mesh=_MESH,
    compiler_params=pltpu.CompilerParams(needs_layout_passes=False),
    scratch_types=[
        pltpu.VMEM((Hb, Wb), jnp.float32),
        pltpu.VMEM((4, Wr), jnp.int32),
        pltpu.VMEM((4, Wr), jnp.float32),
        pltpu.VMEM((2, TPC, 8, 128), jnp.float32),
        pltpu.SemaphoreType.DMA,
        pltpu.SemaphoreType.DMA,
    ],
)
def _sc_sample(bev_hbm, idx_hbm, w_hbm, out_hbm, plane_v, idx_v, w_v, rep_v,
               sem0, sem1):
    wid = lax.axis_index("s") * NC + lax.axis_index("c")
    sems = (sem0, sem1)
    pltpu.sync_copy(idx_hbm, idx_v)
    pltpu.sync_copy(w_hbm, w_v)

    pending = [None, None]
    for p in range(PPW):
        plane = wid * PPW + p
        pltpu.sync_copy(bev_hbm.at[plane], plane_v)
        buf = p % 2
        if pending[buf] is not None:
            for h in pending[buf]:
                h.wait()

        def tile(xt, carry):
            def chunk(jj, carry2):
                j = xt * 8 + jj
                s = pl.ds(pl.multiple_of(j * L, L), L)
                y0 = idx_v[0, s]
                y1 = idx_v[1, s]
                x0 = idx_v[2, s]
                x1 = idx_v[3, s]
                v00 = plsc.load_gather(plane_v, [y0, x0])
                v01 = plsc.load_gather(plane_v, [y0, x1])
                v10 = plsc.load_gather(plane_v, [y1, x0])
                v11 = plsc.load_gather(plane_v, [y1, x1])
                acc = (v00 * w_v[0, s] + v01 * w_v[1, s]
                       + v10 * w_v[2, s] + v11 * w_v[3, s])
                cs = pl.ds(pl.multiple_of(jj * L, L), L)
                for r in range(8):
                    rep_v[buf, xt, r, cs] = acc
                return carry2

            lax.fori_loop(0, 8, chunk, 0)
            return carry

        lax.fori_loop(0, TPC, tile, 0)
        pending[buf] = [
            pltpu.async_copy(rep_v.at[buf], out_hbm.at[plane, rb], sems[buf])
            for rb in range(Hr // 8)
        ]
    for hs in pending:
        if hs is not None:
            for h in hs:
                h.wait()


def kernel(bev_feat, ref_rv):
    B, C = ref_rv.shape[0], ref_rv.shape[1]
    planes = bev_feat.reshape(PLANES, Hb, Wb)
    out = _sc_sample(planes, jnp.asarray(_IDX), jnp.asarray(_W))
    return out.reshape(B, C, Hr, Wr)


# speedup vs baseline: 2.3668x; 2.3668x over previous
import functools

import jax
import jax.numpy as jnp
import numpy as np
from jax import lax
from jax.experimental import pallas as pl
from jax.experimental.pallas import tpu as pltpu
from jax.experimental.pallas import tpu_sc as plsc

Hr, Wr = 64, 2048
Hb, Wb = 256, 256
R_MAX = 51.2

NC, NS, L = 2, 16, 16
NW = NC * NS
PLANES = 4 * 64
PPW = PLANES // NW
CHUNKS = Wr // L


def _constants():
    col = np.arange(Wr, dtype=np.float64)
    phi = (Wr - 1 - col) / (Wr - 1) * 2.0 * np.pi
    x = R_MAX * np.cos(phi)
    y = R_MAX * np.sin(phi)
    idx_x = x / R_MAX * (Wb / 2 - 0.5) + (Wb / 2 - 0.5)
    idx_y = y / R_MAX * (Hb / 2 - 0.5) + (Hb / 2 - 0.5)
    gx = (idx_x / Wb * 2.0 - 1.0).astype(np.float32)
    gy = (idx_y / Hb * 2.0 - 1.0).astype(np.float32)
    one, half = np.float32(1.0), np.float32(0.5)
    ix = (gx + one) * half * np.float32(Wb - 1)
    iy = (gy + one) * half * np.float32(Hb - 1)
    ix0 = np.floor(ix)
    iy0 = np.floor(iy)
    wx1 = ix - ix0
    wx0 = one - wx1
    wy1 = iy - iy0
    wy0 = one - wy1
    ix0i = ix0.astype(np.int32)
    iy0i = iy0.astype(np.int32)
    idx = np.stack([iy0i, iy0i + 1, ix0i, ix0i + 1])
    w = np.stack([wy0 * wx0, wy0 * wx1, wy1 * wx0, wy1 * wx1])
    return idx.astype(np.int32), w.astype(np.float32)


_IDX, _W = _constants()
_MESH = plsc.VectorSubcoreMesh(core_axis_name="c", subcore_axis_name="s")


@functools.partial(
    pl.kernel,
    out_type=jax.ShapeDtypeStruct((PLANES, Hr, Wr), jnp.float32),
    mesh=_MESH,
    compiler_params=pltpu.CompilerParams(needs_layout_passes=False),
    scratch_types=[
        pltpu.VMEM((Hb, Wb), jnp.float32),
        pltpu.VMEM((4, Wr), jnp.int32),
        pltpu.VMEM((4, Wr), jnp.float32),
        pltpu.VMEM((2, 8, Wr), jnp.float32),
        pltpu.SemaphoreType.DMA,
        pltpu.SemaphoreType.DMA,
    ],
)
def _sc_sample(bev_hbm, idx_hbm, w_hbm, out_hbm, plane_v, idx_v, w_v, rep_v,
               sem0, sem1):
    wid = lax.axis_index("s") * NC + lax.axis_index("c")
    sems = (sem0, sem1)
    pltpu.sync_copy(idx_hbm, idx_v)
    pltpu.sync_copy(w_hbm, w_v)

    pending = [None, None]
    for p in range(PPW):
        plane = wid * PPW + p
        pltpu.sync_copy(bev_hbm.at[plane], plane_v)
        buf = p % 2
        if pending[buf] is not None:
            for h in pending[buf]:
                h.wait()

        def chunk(j, carry2):
            s = pl.ds(pl.multiple_of(j * L, L), L)
            y0 = idx_v[0, s]
            y1 = idx_v[1, s]
            x0 = idx_v[2, s]
            x1 = idx_v[3, s]
            v00 = plsc.load_gather(plane_v, [y0, x0])
            v01 = plsc.load_gather(plane_v, [y0, x1])
            v10 = plsc.load_gather(plane_v, [y1, x0])
            v11 = plsc.load_gather(plane_v, [y1, x1])
            acc = (v00 * w_v[0, s] + v01 * w_v[1, s]
                   + v10 * w_v[2, s] + v11 * w_v[3, s])
            for r in range(8):
                rep_v[buf, r, s] = acc
            return carry2

        lax.fori_loop(0, CHUNKS, chunk, 0)
        pending[buf] = [
            pltpu.async_copy(rep_v.at[buf], out_hbm.at[plane, pl.ds(rb * 8, 8)],
                             sems[buf])
            for rb in range(Hr // 8)
        ]
    for hs in pending:
        if hs is not None:
            for h in hs:
                h.wait()


def kernel(bev_feat, ref_rv):
    B, C = ref_rv.shape[0], ref_rv.shape[1]
    planes = bev_feat.reshape(PLANES, Hb, Wb)
    out = _sc_sample(planes, jnp.asarray(_IDX), jnp.asarray(_W))
    return out.reshape(B, C, Hr, Wr)
